# baseline (speedup 1.0000x reference)
import jax
import jax.numpy as jnp
from jax.experimental import pallas as pl
from jax.experimental.pallas import tpu as pltpu

N = 1024
B = 8
P = 32768
F = 16
HID = 64


def _diag_body(nf_ref, ed_ref, w1a_ref, w1b_ref, b1_ref, w2_ref, b2_ref,
               ab_ref, out_ref):
    h = jnp.tanh(nf_ref[...] @ w1a_ref[...]
                 + ed_ref[...] * w1b_ref[...]
                 + b1_ref[...])
    out_ref[...] = (h @ w2_ref[...] + b2_ref[...]) * ab_ref[...]


def _off_body(g_ref, w2_ref, b2_ref, ob_ref, out_ref):
    h = jnp.tanh(g_ref[...])
    out_ref[...] = (h @ w2_ref[...] + b2_ref[...]) * ob_ref[...]


def _base_body(dg_ref, out_ref):
    r = pl.program_id(0)
    off = (r % 16) * B
    bi = jax.lax.broadcasted_iota(jnp.int32, (B, 128), 0)
    ci = jax.lax.broadcasted_iota(jnp.int32, (B, 128), 1)
    E = (ci == off + bi).astype(jnp.float32)
    tile = dg_ref[0] @ E
    out_ref[...] = jnp.zeros_like(out_ref)
    out_ref[:, pl.ds(pl.multiple_of((r // 16) * 128, 128), 128)] = tile


def kernel(nodes_features, connectivity_mask, atom_blocks, off_diag_blocks,
           W_d1, b_d1, W_d2, b_d2, W_o1, b_o1, W_o2, b_o2, pair_index):
    i = pair_index[:, 0]
    j = pair_index[:, 1]
    edge_diag = jnp.diagonal(connectivity_mask)[:, None]

    diag_blk = pl.pallas_call(
        _diag_body,
        out_shape=jax.ShapeDtypeStruct((N, B * B), jnp.float32),
    )(nodes_features, edge_diag, W_d1[:F], W_d1[F][None, :], b_d1[None, :],
      W_d2, b_d2[None, :], atom_blocks.reshape(N, B * B))

    A = nodes_features @ W_o1[:F]
    Bm = nodes_features @ W_o1[F:2 * F]
    e_ij = connectivity_mask[i, j][:, None]
    G = A[i] + Bm[j] + e_ij * W_o1[2 * F][None, :] + b_o1[None, :]

    BP = 4096
    off_flat = pl.pallas_call(
        _off_body,
        grid=(P // BP,),
        in_specs=[
            pl.BlockSpec((BP, HID), lambda k: (k, 0)),
            pl.BlockSpec((HID, B * B), lambda k: (0, 0)),
            pl.BlockSpec((1, B * B), lambda k: (0, 0)),
            pl.BlockSpec((BP, B * B), lambda k: (k, 0)),
        ],
        out_specs=pl.BlockSpec((BP, B * B), lambda k: (k, 0)),
        out_shape=jax.ShapeDtypeStruct((P, B * B), jnp.float32),
    )(G, W_o2, b_o2[None, :], off_diag_blocks.reshape(P, B * B))
    off_blk = off_flat.reshape(P, B, B)

    H_base = pl.pallas_call(
        _base_body,
        grid=(N,),
        in_specs=[pl.BlockSpec((1, B, B), lambda r: (r, 0, 0))],
        out_specs=pl.BlockSpec((B, N * B), lambda r: (r, 0)),
        out_shape=jax.ShapeDtypeStruct((N * B, N * B), jnp.float32),
    )(diag_blk.reshape(N, B, B))

    ar = jnp.arange(B)
    r_o = (i * B)[:, None, None] + ar[None, :, None]
    c_o = (j * B)[:, None, None] + ar[None, None, :]
    H = H_base.at[r_o, c_o].set(off_blk)
    r_s = (j * B)[:, None, None] + ar[None, :, None]
    c_s = (i * B)[:, None, None] + ar[None, None, :]
    H = H.at[r_s, c_s].set(jnp.swapaxes(off_blk, 1, 2))
    return H


# speedup vs baseline: 1.0083x; 1.0083x over previous
import jax
import jax.numpy as jnp
from jax import lax
from jax.experimental import pallas as pl
from jax.experimental.pallas import tpu as pltpu
from jax.experimental.pallas import tpu_sc as plsc

N = 1024
B = 8
P = 32768
F = 16
HID = 64

NC = 2
NS = 16
NW = NC * NS
PPW = P // NW
CH = 128
NCH = PPW // CH


def _diag_body(nf_ref, ed_ref, w1a_ref, w1b_ref, b1_ref, w2_ref, b2_ref,
               ab_ref, wo1a_ref, wo1b_ref, out_ref, tab_ref):
    h = jnp.tanh(nf_ref[...] @ w1a_ref[...]
                 + ed_ref[...] * w1b_ref[...]
                 + b1_ref[...])
    out_ref[...] = (h @ w2_ref[...] + b2_ref[...]) * ab_ref[...]
    tab_ref[...] = jnp.concatenate(
        [nf_ref[...] @ wo1a_ref[...], nf_ref[...] @ wo1b_ref[...]], axis=1)


def _off_body(g_ref, er_ref, lane_ref, ew_ref, b1_ref, w2_ref, b2_ref,
              ob_ref, out_ref):
    li = lax.broadcasted_iota(jnp.int32, er_ref.shape, 1)
    onehot = (li == lane_ref[...]).astype(jnp.float32)
    e = jnp.sum(er_ref[...] * onehot, axis=1, keepdims=True)
    h = jnp.tanh(g_ref[...] + e * ew_ref[...] + b1_ref[...])
    out_ref[...] = (h @ w2_ref[...] + b2_ref[...]) * ob_ref[...]


def _base_body(dg_ref, out_ref):
    r = pl.program_id(0)
    off = (r % 16) * B
    bi = lax.broadcasted_iota(jnp.int32, (B, 128), 0)
    ci = lax.broadcasted_iota(jnp.int32, (B, 128), 1)
    sel = (ci == off + bi).astype(jnp.float32)
    tile = dg_ref[0] @ sel
    out_ref[...] = jnp.zeros_like(out_ref)
    out_ref[:, pl.ds(pl.multiple_of((r // 16) * 128, 128), 128)] = tile


def _gather_kernel(tab_hbm, cm8_hbm, i_hbm, j_hbm,
                   out_hbm, er_out_hbm,
                   idx_i, idx_j, idx_e, abi, abj, erows, gout, sem):
    wid = lax.axis_index("s") * NC + lax.axis_index("c")

    def chunk_body(k, carry):
        base = wid * PPW + k * CH
        pltpu.sync_copy(i_hbm.at[pl.ds(base, CH)], idx_i)
        pltpu.sync_copy(j_hbm.at[pl.ds(base, CH)], idx_j)
        for t in range(CH // 16):
            iv = idx_i[pl.ds(t * 16, 16)]
            jv = idx_j[pl.ds(t * 16, 16)]
            idx_e[pl.ds(t * 16, 16)] = lax.shift_right_logical(iv * N + jv, 7)
        pltpu.async_copy(tab_hbm.at[idx_i], abi, sem).wait()
        pltpu.async_copy(tab_hbm.at[idx_j], abj, sem).wait()
        pltpu.async_copy(cm8_hbm.at[idx_e], erows, sem).wait()

        def row_body(rr, carry2):
            for q in range(HID // 16):
                gout[rr, pl.ds(q * 16, 16)] = (
                    abi[rr, pl.ds(q * 16, 16)]
                    + abj[rr, pl.ds(HID + q * 16, 16)])
            return carry2

        lax.fori_loop(0, CH, row_body, 0)
        pltpu.sync_copy(gout, out_hbm.at[pl.ds(base, CH)])
        pltpu.sync_copy(erows, er_out_hbm.at[pl.ds(base, CH)])
        return carry

    lax.fori_loop(0, NCH, chunk_body, 0)


def _gather_G(tab, cm8, i, j):
    mesh = plsc.VectorSubcoreMesh(core_axis_name="c", subcore_axis_name="s")
    return pl.kernel(
        _gather_kernel,
        mesh=mesh,
        out_type=[
            jax.ShapeDtypeStruct((P, HID), jnp.float32),
            jax.ShapeDtypeStruct((P, 128), jnp.float32),
        ],
        scratch_types=[
            pltpu.VMEM((CH,), jnp.int32),
            pltpu.VMEM((CH,), jnp.int32),
            pltpu.VMEM((CH,), jnp.int32),
            pltpu.VMEM((CH, 128), jnp.float32),
            pltpu.VMEM((CH, 128), jnp.float32),
            pltpu.VMEM((CH, 128), jnp.float32),
            pltpu.VMEM((CH, HID), jnp.float32),
            pltpu.SemaphoreType.DMA,
        ],
    )(tab, cm8, i, j)


def kernel(nodes_features, connectivity_mask, atom_blocks, off_diag_blocks,
           W_d1, b_d1, W_d2, b_d2, W_o1, b_o1, W_o2, b_o2, pair_index):
    i = pair_index[:, 0]
    j = pair_index[:, 1]
    edge_diag = jnp.diagonal(connectivity_mask)[:, None]

    diag_blk, tab = pl.pallas_call(
        _diag_body,
        out_shape=[
            jax.ShapeDtypeStruct((N, B * B), jnp.float32),
            jax.ShapeDtypeStruct((N, 2 * HID), jnp.float32),
        ],
    )(nodes_features, edge_diag, W_d1[:F], W_d1[F][None, :], b_d1[None, :],
      W_d2, b_d2[None, :], atom_blocks.reshape(N, B * B),
      W_o1[:F], W_o1[F:2 * F])

    cm8 = connectivity_mask.reshape(N * N // 128, 128)
    G, erows = _gather_G(tab, cm8, i, j)
    lane = ((i * N + j) & 127).astype(jnp.int32)[:, None]

    BP = 4096
    off_flat = pl.pallas_call(
        _off_body,
        grid=(P // BP,),
        in_specs=[
            pl.BlockSpec((BP, HID), lambda k: (k, 0)),
            pl.BlockSpec((BP, 128), lambda k: (k, 0)),
            pl.BlockSpec((BP, 1), lambda k: (k, 0)),
            pl.BlockSpec((1, HID), lambda k: (0, 0)),
            pl.BlockSpec((1, HID), lambda k: (0, 0)),
            pl.BlockSpec((HID, B * B), lambda k: (0, 0)),
            pl.BlockSpec((1, B * B), lambda k: (0, 0)),
            pl.BlockSpec((BP, B * B), lambda k: (k, 0)),
        ],
        out_specs=pl.BlockSpec((BP, B * B), lambda k: (k, 0)),
        out_shape=jax.ShapeDtypeStruct((P, B * B), jnp.float32),
    )(G, erows, lane, W_o1[2 * F][None, :], b_o1[None, :],
      W_o2, b_o2[None, :], off_diag_blocks.reshape(P, B * B))
    off_blk = off_flat.reshape(P, B, B)

    H_base = pl.pallas_call(
        _base_body,
        grid=(N,),
        in_specs=[pl.BlockSpec((1, B, B), lambda r: (r, 0, 0))],
        out_specs=pl.BlockSpec((B, N * B), lambda r: (r, 0)),
        out_shape=jax.ShapeDtypeStruct((N * B, N * B), jnp.float32),
    )(diag_blk.reshape(N, B, B))

    ar = jnp.arange(B)
    r_o = (i * B)[:, None, None] + ar[None, :, None]
    c_o = (j * B)[:, None, None] + ar[None, None, :]
    H = H_base.at[r_o, c_o].set(off_blk)
    r_s = (j * B)[:, None, None] + ar[None, :, None]
    c_s = (i * B)[:, None, None] + ar[None, None, :]
    H = H.at[r_s, c_s].set(jnp.swapaxes(off_blk, 1, 2))
    return H


# speedup vs baseline: 1.0297x; 1.0212x over previous
import jax
import jax.numpy as jnp
from jax import lax
from jax.experimental import pallas as pl
from jax.experimental.pallas import tpu as pltpu
from jax.experimental.pallas import tpu_sc as plsc

N = 1024
B = 8
P = 32768
F = 16
HID = 64

NC = 2
NS = 16
NW = NC * NS
PPW = P // NW
CH = 128
NCH = PPW // CH


def _diag_body(nf_ref, ed_ref, w1a_ref, w1b_ref, b1_ref, w2_ref, b2_ref,
               ab_ref, wo1a_ref, wo1b_ref, out_ref, tab_ref):
    h = jnp.tanh(nf_ref[...] @ w1a_ref[...]
                 + ed_ref[...] * w1b_ref[...]
                 + b1_ref[...])
    out_ref[...] = (h @ w2_ref[...] + b2_ref[...]) * ab_ref[...]
    tab_ref[...] = jnp.concatenate(
        [nf_ref[...] @ wo1a_ref[...], nf_ref[...] @ wo1b_ref[...]], axis=1)


def _off_body(g_ref, er_ref, lane_ref, ew_ref, b1_ref, w2_ref, b2_ref,
              ob_ref, out_ref):
    li = lax.broadcasted_iota(jnp.int32, er_ref.shape, 1)
    onehot = (li == lane_ref[...]).astype(jnp.float32)
    e = jnp.sum(er_ref[...] * onehot, axis=1, keepdims=True)
    h = jnp.tanh(g_ref[...] + e * ew_ref[...] + b1_ref[...])
    out_ref[...] = (h @ w2_ref[...] + b2_ref[...]) * ob_ref[...]


def _base_body(dg_ref, out_ref):
    s = pl.program_id(0)
    tall = dg_ref[...]
    bi = lax.broadcasted_iota(jnp.int32, (B, 128), 0)
    ci = lax.broadcasted_iota(jnp.int32, (B, 128), 1)
    rep = ((ci % B) == bi).astype(jnp.float32)
    ri2 = lax.broadcasted_iota(jnp.int32, (128, 128), 0)
    ci2 = lax.broadcasted_iota(jnp.int32, (128, 128), 1)
    tile = (tall @ rep) * (ri2 // B == ci2 // B).astype(jnp.float32)
    out_ref[...] = jnp.zeros_like(out_ref)
    out_ref[:, pl.ds(pl.multiple_of(s * 128, 128), 128)] = tile


def _gather_kernel(tab_hbm, cm8_hbm, i_hbm, j_hbm,
                   out_hbm, er_out_hbm,
                   idx_i, idx_j, idx_e, abi, abj, erows, gout, sem):
    wid = lax.axis_index("s") * NC + lax.axis_index("c")

    def chunk_body(k, carry):
        base = wid * PPW + k * CH
        pltpu.sync_copy(i_hbm.at[pl.ds(base, CH)], idx_i)
        pltpu.sync_copy(j_hbm.at[pl.ds(base, CH)], idx_j)
        for t in range(CH // 16):
            iv = idx_i[pl.ds(t * 16, 16)]
            jv = idx_j[pl.ds(t * 16, 16)]
            idx_e[pl.ds(t * 16, 16)] = lax.shift_right_logical(iv * N + jv, 7)
        pltpu.async_copy(tab_hbm.at[idx_i], abi, sem).wait()
        pltpu.async_copy(tab_hbm.at[idx_j], abj, sem).wait()
        pltpu.async_copy(cm8_hbm.at[idx_e], erows, sem).wait()

        def row_body(rr, carry2):
            for q in range(HID // 16):
                gout[rr, pl.ds(q * 16, 16)] = (
                    abi[rr, pl.ds(q * 16, 16)]
                    + abj[rr, pl.ds(HID + q * 16, 16)])
            return carry2

        lax.fori_loop(0, CH, row_body, 0)
        pltpu.sync_copy(gout, out_hbm.at[pl.ds(base, CH)])
        pltpu.sync_copy(erows, er_out_hbm.at[pl.ds(base, CH)])
        return carry

    lax.fori_loop(0, NCH, chunk_body, 0)


def _gather_G(tab, cm8, i, j):
    mesh = plsc.VectorSubcoreMesh(core_axis_name="c", subcore_axis_name="s")
    return pl.kernel(
        _gather_kernel,
        mesh=mesh,
        out_type=[
            jax.ShapeDtypeStruct((P, HID), jnp.float32),
            jax.ShapeDtypeStruct((P, 128), jnp.float32),
        ],
        scratch_types=[
            pltpu.VMEM((CH,), jnp.int32),
            pltpu.VMEM((CH,), jnp.int32),
            pltpu.VMEM((CH,), jnp.int32),
            pltpu.VMEM((CH, 128), jnp.float32),
            pltpu.VMEM((CH, 128), jnp.float32),
            pltpu.VMEM((CH, 128), jnp.float32),
            pltpu.VMEM((CH, HID), jnp.float32),
            pltpu.SemaphoreType.DMA,
        ],
    )(tab, cm8, i, j)


def kernel(nodes_features, connectivity_mask, atom_blocks, off_diag_blocks,
           W_d1, b_d1, W_d2, b_d2, W_o1, b_o1, W_o2, b_o2, pair_index):
    i = pair_index[:, 0]
    j = pair_index[:, 1]
    edge_diag = jnp.diagonal(connectivity_mask)[:, None]

    diag_blk, tab = pl.pallas_call(
        _diag_body,
        out_shape=[
            jax.ShapeDtypeStruct((N, B * B), jnp.float32),
            jax.ShapeDtypeStruct((N, 2 * HID), jnp.float32),
        ],
    )(nodes_features, edge_diag, W_d1[:F], W_d1[F][None, :], b_d1[None, :],
      W_d2, b_d2[None, :], atom_blocks.reshape(N, B * B),
      W_o1[:F], W_o1[F:2 * F])

    cm8 = connectivity_mask.reshape(N * N // 128, 128)
    G, erows = _gather_G(tab, cm8, i, j)
    lane = ((i * N + j) & 127).astype(jnp.int32)[:, None]

    BP = 4096
    off_flat = pl.pallas_call(
        _off_body,
        grid=(P // BP,),
        in_specs=[
            pl.BlockSpec((BP, HID), lambda k: (k, 0)),
            pl.BlockSpec((BP, 128), lambda k: (k, 0)),
            pl.BlockSpec((BP, 1), lambda k: (k, 0)),
            pl.BlockSpec((1, HID), lambda k: (0, 0)),
            pl.BlockSpec((1, HID), lambda k: (0, 0)),
            pl.BlockSpec((HID, B * B), lambda k: (0, 0)),
            pl.BlockSpec((1, B * B), lambda k: (0, 0)),
            pl.BlockSpec((BP, B * B), lambda k: (k, 0)),
        ],
        out_specs=pl.BlockSpec((BP, B * B), lambda k: (k, 0)),
        out_shape=jax.ShapeDtypeStruct((P, B * B), jnp.float32),
    )(G, erows, lane, W_o1[2 * F][None, :], b_o1[None, :],
      W_o2, b_o2[None, :], off_diag_blocks.reshape(P, B * B))
    off_blk = off_flat.reshape(P, B, B)

    H_base = pl.pallas_call(
        _base_body,
        grid=(N * B // 128,),
        in_specs=[pl.BlockSpec((128, B), lambda s: (s, 0))],
        out_specs=pl.BlockSpec((128, N * B), lambda s: (s, 0)),
        out_shape=jax.ShapeDtypeStruct((N * B, N * B), jnp.float32),
    )(diag_blk.reshape(N * B, B))

    ar = jnp.arange(B)
    r_o = (i * B)[:, None, None] + ar[None, :, None]
    c_o = (j * B)[:, None, None] + ar[None, None, :]
    H = H_base.at[r_o, c_o].set(off_blk)
    r_s = (j * B)[:, None, None] + ar[None, :, None]
    c_s = (i * B)[:, None, None] + ar[None, None, :]
    H = H.at[r_s, c_s].set(jnp.swapaxes(off_blk, 1, 2))
    return H
